# Initial kernel scaffold; baseline (speedup 1.0000x reference)
import jax
import jax.numpy as jnp
from jax.experimental import pallas as pl


def kernel(inputs, targets):
    raise NotImplementedError("write your pallas kernel here")


# speedup vs baseline: 17.9742x; 17.9742x over previous
import functools

import jax
import jax.numpy as jnp
from jax import lax
from jax.experimental import pallas as pl
from jax.experimental.pallas import tpu as pltpu
from jax.experimental.pallas import tpu_sc as plsc

SHIFT = 14
NB = 1 << (31 - SHIFT)
PLANES = 2 * NB
NIMG = 16
NPIX = 512 * 512
NC, NS, LANES = 2, 16, 16
PER_TILE = NPIX // NS
VECS = PER_TILE // LANES
ROWS = PER_TILE // 128
IMGS_PER_SC = NIMG // NC

R2, C2 = 512, 256


def _sc_histogram_body(x_hbm, y_hbm, out_hbm, acc, x_v, y_v, idx_v, ones_v, zero_v):
    c = lax.axis_index("c")
    s = lax.axis_index("s")
    lane = lax.iota(jnp.int32, LANES)

    for k in range(8):
        ones_v[pl.ds(k * LANES, LANES)] = jnp.full((LANES,), 1.0, jnp.float32)

    def zfill(i, _):
        zero_v[pl.ds(i * LANES, LANES)] = jnp.zeros((LANES,), jnp.float32)
        return 0
    lax.fori_loop(0, 2048 // LANES, zfill, 0)

    def zcopy(i, _):
        pltpu.sync_copy(zero_v, acc.at[pl.ds(s * (PLANES // NS) + i * 2048, 2048)])
        return 0
    lax.fori_loop(0, PLANES // NS // 2048, zcopy, 0)
    plsc.subcore_barrier()

    for imgi in range(IMGS_PER_SC):
        img = c * IMGS_PER_SC + imgi
        base = s * PER_TILE
        pltpu.sync_copy(x_hbm.at[img, pl.ds(base, PER_TILE)], x_v)
        pltpu.sync_copy(y_hbm.at[img, pl.ds(base, PER_TILE)], y_v)

        def compute(j, _):
            for k in range(VECS // ROWS):
                off = j * 128 + k * LANES
                xv = x_v[pl.ds(off, LANES)]
                yv = y_v[pl.ds(off, LANES)]
                yf = yv.astype(jnp.float32)
                e = 1.0 - xv * (2.0 * yf - 1.0)
                bits = lax.bitcast_convert_type(e, jnp.int32)
                bucket = lax.shift_right_logical(bits, SHIFT)
                m = lax.shift_right_arithmetic(bits, 31)
                idx = ((bucket + lax.shift_left(yv, 31 - SHIFT)) & ~m) | (lane & m)
                idx_v[j, pl.ds(k * LANES, LANES)] = idx
            return 0
        lax.fori_loop(0, ROWS, compute, 0)

        def scatter(j, _):
            pltpu.sync_copy(ones_v, acc.at[idx_v.at[j]], add=True)
            return 0
        lax.fori_loop(0, ROWS, scatter, 0)
        plsc.subcore_barrier()

        pltpu.sync_copy(acc.at[pl.ds(s * (PLANES // NS), PLANES // NS)],
                        out_hbm.at[img, pl.ds(s * (PLANES // NS), PLANES // NS)])
        plsc.subcore_barrier()


@functools.cache
def _sc_histogram():
    return pl.kernel(
        _sc_histogram_body,
        out_type=jax.ShapeDtypeStruct((NIMG, PLANES), jnp.float32),
        mesh=plsc.VectorSubcoreMesh(core_axis_name="c", subcore_axis_name="s",
                                    num_cores=NC, num_subcores=NS),
        scratch_types=[
            pltpu.VMEM_SHARED((PLANES,), jnp.float32),
            pltpu.VMEM((PER_TILE,), jnp.float32),
            pltpu.VMEM((PER_TILE,), jnp.int32),
            pltpu.VMEM((ROWS, 128), jnp.int32),
            pltpu.VMEM((128,), jnp.float32),
            pltpu.VMEM((2048,), jnp.float32),
        ],
    )


def _tc_loss_body(t_ref, tprev_ref, tgt_ref, out_ref):
    i = pl.program_id(0)
    first = jnp.logical_or(i == 0, i == IMGS_PER_SC)
    cur = t_ref[0]
    prev = jnp.where(first, 0.0, tprev_ref[0])
    d = cur - prev
    h = d[0] + d[1]
    p = d[1]

    G = jnp.sum(tgt_ref[0]).astype(jnp.float32)

    iu = lax.broadcasted_iota(jnp.int32, (C2, C2), 0)
    ju = lax.broadcasted_iota(jnp.int32, (C2, C2), 1)
    umat = (iu <= ju).astype(jnp.float32)
    il = lax.broadcasted_iota(jnp.int32, (R2, R2), 0)
    jl = lax.broadcasted_iota(jnp.int32, (R2, R2), 1)
    lmat = (il > jl).astype(jnp.float32)

    def flat_prefix(a):
        rowp = jnp.dot(a, umat, preferred_element_type=jnp.float32)
        carry = jnp.dot(lmat, rowp[:, C2 - 1:C2],
                        preferred_element_type=jnp.float32)
        return rowp + carry

    ph = flat_prefix(h)
    pp = flat_prefix(p)
    th = ph[R2 - 1:R2, C2 - 1:C2]
    tp = pp[R2 - 1:R2, C2 - 1:C2]
    n_incl = th - ph + h
    n_excl = th - ph
    c_incl = tp - pp + p
    c_excl = tp - pp

    def jac(n, cc):
        denom = jnp.maximum(G + n - cc, 1.0)
        return jnp.where(n > 0.0, 1.0 - (G - cc) / denom, 0.0)

    b2 = (lax.broadcasted_iota(jnp.int32, (R2, C2), 0) * C2
          + lax.broadcasted_iota(jnp.int32, (R2, C2), 1))
    rep = lax.bitcast_convert_type(
        lax.shift_left(b2, SHIFT) + (1 << (SHIFT - 1)), jnp.float32)
    rep = jnp.where(jnp.logical_or(b2 < LANES, b2 >= (255 << (23 - SHIFT))),
                    0.0, rep)
    loss = jnp.sum(rep * (jac(n_incl, c_incl) - jac(n_excl, c_excl)))

    @pl.when(i == 0)
    def _():
        out_ref[...] = jnp.zeros((1, 1), jnp.float32)
    out_ref[...] += jnp.reshape(loss / NIMG, (1, 1))


@functools.partial(jax.jit, static_argnames=())
def kernel(inputs, targets):
    x = inputs.reshape(NIMG, NPIX)
    y = targets.reshape(NIMG, NPIX)
    t = _sc_histogram()(x, y)
    t4 = t.reshape(NIMG, 2, R2, C2)
    loss = pl.pallas_call(
        _tc_loss_body,
        grid=(NIMG,),
        in_specs=[
            pl.BlockSpec((1, 2, R2, C2), lambda i: (i, 0, 0, 0)),
            pl.BlockSpec((1, 2, R2, C2), lambda i: (jnp.maximum(i - 1, 0), 0, 0, 0)),
            pl.BlockSpec((1, 512, 512), lambda i: (i, 0, 0)),
        ],
        out_specs=pl.BlockSpec((1, 1), lambda i: (0, 0)),
        out_shape=jax.ShapeDtypeStruct((1, 1), jnp.float32),
    )(t4, t4, targets)
    return loss[0, 0]
